# baseline (speedup 1.0000x reference)
import jax
import jax.numpy as jnp
from jax.experimental import pallas as pl
from jax.experimental.pallas import tpu as pltpu

DEPTH_ = 100000
BATCH_ = 1024

ZERO_END = 99328
BLOCK_C = 1024
N_ZCOPY = ZERO_END // BLOCK_C
TAIL_C = 1024
N_TBLK = ZERO_END // TAIL_C


def _fill_body(idx_v_ref, idx_s_ref, out_hbm, zbuf, pb, zsem, psem):
    idx = idx_v_ref[...]
    lane = jax.lax.broadcasted_iota(jnp.int32, (BATCH_, 128), 1)

    zbuf[...] = jnp.zeros((BATCH_, BLOCK_C), jnp.float32)
    pb[...] = (lane == jax.lax.rem(idx, 128)).astype(jnp.float32)

    def _zstart(i, _):
        pltpu.make_async_copy(
            zbuf, out_hbm.at[:, pl.ds(i * BLOCK_C, BLOCK_C)], zsem
        ).start()
        return 0

    jax.lax.fori_loop(0, N_ZCOPY, _zstart, 0)

    def _zwait(i, _):
        pltpu.make_async_copy(
            zbuf, out_hbm.at[:, pl.ds(i * BLOCK_C, BLOCK_C)], zsem
        ).wait()
        return 0

    jax.lax.fori_loop(0, N_ZCOPY, _zwait, 0)

    def _patch(r, _):
        idxr = idx_s_ref[r, 0]
        c0 = pl.multiple_of((idxr // 128) * 128, 128)

        @pl.when(idxr < ZERO_END)
        def _():
            pltpu.make_async_copy(
                pb.at[pl.ds(r, 1), :],
                out_hbm.at[pl.ds(r, 1), pl.ds(c0, 128)],
                psem,
            ).start()

        return 0

    jax.lax.fori_loop(0, BATCH_, _patch, 0)

    def _pwait(r, _):
        idxr = idx_s_ref[r, 0]
        c0 = pl.multiple_of((idxr // 128) * 128, 128)

        @pl.when(idxr < ZERO_END)
        def _():
            pltpu.make_async_copy(
                pb.at[pl.ds(r, 1), :],
                out_hbm.at[pl.ds(r, 1), pl.ds(c0, 128)],
                psem,
            ).wait()

        return 0

    jax.lax.fori_loop(0, BATCH_, _pwait, 0)


def _tail_body(prev_ref, idx_ref, col_ref, out_ref):
    del prev_ref
    idx = idx_ref[...]
    col = col_ref[...]
    out_ref[...] = (col == idx).astype(jnp.float32)


def kernel(input):
    idx = input.astype(jnp.int32)
    filled = pl.pallas_call(
        _fill_body,
        in_specs=[
            pl.BlockSpec(memory_space=pltpu.VMEM),
            pl.BlockSpec(memory_space=pltpu.SMEM),
        ],
        out_specs=pl.BlockSpec(memory_space=pl.ANY),
        out_shape=jax.ShapeDtypeStruct((BATCH_, DEPTH_), jnp.float32),
        scratch_shapes=[
            pltpu.VMEM((BATCH_, BLOCK_C), jnp.float32),
            pltpu.VMEM((BATCH_, 128), jnp.float32),
            pltpu.SemaphoreType.DMA,
            pltpu.SemaphoreType.DMA,
        ],
    )(idx, idx)

    col = jax.lax.broadcasted_iota(jnp.int32, (1, DEPTH_), 1)
    out = pl.pallas_call(
        _tail_body,
        grid=(1,),
        in_specs=[
            pl.BlockSpec(memory_space=pl.ANY),
            pl.BlockSpec((BATCH_, 1), lambda i: (0, 0)),
            pl.BlockSpec((1, TAIL_C), lambda i: (0, N_TBLK)),
        ],
        out_specs=pl.BlockSpec((BATCH_, TAIL_C), lambda i: (0, N_TBLK)),
        out_shape=jax.ShapeDtypeStruct((BATCH_, DEPTH_), jnp.float32),
        input_output_aliases={0: 0},
    )(filled, idx, col)
    return out


# speedup vs baseline: 1.0310x; 1.0310x over previous
import jax
import jax.numpy as jnp
from jax.experimental import pallas as pl
from jax.experimental.pallas import tpu as pltpu

DEPTH_ = 100000
BATCH_ = 1024

ZERO_END = 99328
BLOCK_C = 1024
N_ZCOPY = ZERO_END // BLOCK_C
N_OUT = 2
TAIL_C = 1024
N_TBLK = ZERO_END // TAIL_C


def _fill_body(idx_v_ref, idx_s_ref, out_hbm, zbuf, pb, zsem, psem):
    idx = idx_v_ref[...]
    lane = jax.lax.broadcasted_iota(jnp.int32, (BATCH_, 128), 1)

    zbuf[...] = jnp.zeros((BATCH_, BLOCK_C), jnp.float32)
    pb[...] = (lane == jax.lax.rem(idx, 128)).astype(jnp.float32)

    def _zstart(i, _):
        @pl.when(i >= N_OUT)
        def _():
            pltpu.make_async_copy(
                zbuf,
                out_hbm.at[:, pl.ds((i - N_OUT) * BLOCK_C, BLOCK_C)],
                zsem,
            ).wait()

        pltpu.make_async_copy(
            zbuf, out_hbm.at[:, pl.ds(i * BLOCK_C, BLOCK_C)], zsem
        ).start()
        return 0

    jax.lax.fori_loop(0, N_ZCOPY, _zstart, 0)

    def _zwait(i, _):
        pltpu.make_async_copy(
            zbuf,
            out_hbm.at[:, pl.ds((N_ZCOPY - N_OUT + i) * BLOCK_C, BLOCK_C)],
            zsem,
        ).wait()
        return 0

    jax.lax.fori_loop(0, N_OUT, _zwait, 0)

    def _patch(r, _):
        idxr = idx_s_ref[r, 0]
        c0 = pl.multiple_of((idxr // 128) * 128, 128)

        @pl.when(idxr < ZERO_END)
        def _():
            pltpu.make_async_copy(
                pb.at[pl.ds(r, 1), :],
                out_hbm.at[pl.ds(r, 1), pl.ds(c0, 128)],
                psem,
            ).start()

        return 0

    jax.lax.fori_loop(0, BATCH_, _patch, 0)

    def _pwait(r, _):
        idxr = idx_s_ref[r, 0]
        c0 = pl.multiple_of((idxr // 128) * 128, 128)

        @pl.when(idxr < ZERO_END)
        def _():
            pltpu.make_async_copy(
                pb.at[pl.ds(r, 1), :],
                out_hbm.at[pl.ds(r, 1), pl.ds(c0, 128)],
                psem,
            ).wait()

        return 0

    jax.lax.fori_loop(0, BATCH_, _pwait, 0)


def _tail_body(prev_ref, idx_ref, col_ref, out_ref):
    del prev_ref
    idx = idx_ref[...]
    col = col_ref[...]
    out_ref[...] = (col == idx).astype(jnp.float32)


def kernel(input):
    idx = input.astype(jnp.int32)
    filled = pl.pallas_call(
        _fill_body,
        in_specs=[
            pl.BlockSpec(memory_space=pltpu.VMEM),
            pl.BlockSpec(memory_space=pltpu.SMEM),
        ],
        out_specs=pl.BlockSpec(memory_space=pl.ANY),
        out_shape=jax.ShapeDtypeStruct((BATCH_, DEPTH_), jnp.float32),
        scratch_shapes=[
            pltpu.VMEM((BATCH_, BLOCK_C), jnp.float32),
            pltpu.VMEM((BATCH_, 128), jnp.float32),
            pltpu.SemaphoreType.DMA,
            pltpu.SemaphoreType.DMA,
        ],
    )(idx, idx)

    col = jax.lax.broadcasted_iota(jnp.int32, (1, DEPTH_), 1)
    out = pl.pallas_call(
        _tail_body,
        grid=(1,),
        in_specs=[
            pl.BlockSpec(memory_space=pl.ANY),
            pl.BlockSpec((BATCH_, 1), lambda i: (0, 0)),
            pl.BlockSpec((1, TAIL_C), lambda i: (0, N_TBLK)),
        ],
        out_specs=pl.BlockSpec((BATCH_, TAIL_C), lambda i: (0, N_TBLK)),
        out_shape=jax.ShapeDtypeStruct((BATCH_, DEPTH_), jnp.float32),
        input_output_aliases={0: 0},
    )(filled, idx, col)
    return out
